# Initial kernel scaffold; baseline (speedup 1.0000x reference)
import jax
import jax.numpy as jnp
from jax.experimental import pallas as pl


def kernel(Z, Y):
    raise NotImplementedError("write your pallas kernel here")


# speedup vs baseline: 1.7939x; 1.7939x over previous
import functools

import jax
import jax.numpy as jnp
from jax.experimental import pallas as pl
from jax.experimental.pallas import tpu as pltpu

Q = 1024
D = 512
K_TOTAL = 100000
K_TILE = 2000
N_TILES = K_TOTAL // K_TILE
TOPK = 5

NEG_INF = float("-inf")
BIG_I32 = 2**31 - 1


def _merge_candidate(run_vals, run_idx, c_val, c_idx):
    lt = run_vals < c_val
    shifted_vals = jnp.concatenate(
        [jnp.full((Q, 1), jnp.inf, dtype=jnp.float32), run_vals[:, : TOPK - 1]],
        axis=1)
    shifted_idx = jnp.concatenate(
        [jnp.zeros((Q, 1), dtype=jnp.int32), run_idx[:, : TOPK - 1]], axis=1)
    prev_lt = shifted_vals < c_val
    new_vals = jnp.where(prev_lt, shifted_vals, jnp.where(lt, c_val, run_vals))
    new_idx = jnp.where(prev_lt, shifted_idx, jnp.where(lt, c_idx, run_idx))
    return new_vals, new_idx


def _topk_kernel(z_ref, y_ref, vals_out_ref, idx_out_ref, run_vals, run_idx):
    k = pl.program_id(0)

    @pl.when(k == 0)
    def _init():
        run_vals[...] = jnp.full((Q, TOPK), NEG_INF, dtype=jnp.float32)
        run_idx[...] = jnp.zeros((Q, TOPK), dtype=jnp.int32)

    z = z_ref[...]
    y = y_ref[...]
    zn = z / jnp.sqrt(jnp.sum(z * z, axis=1, keepdims=True))
    s = jax.lax.dot_general(
        zn, y,
        dimension_numbers=(((1,), (1,)), ((), ())),
        preferred_element_type=jnp.float32,
    )

    col = jax.lax.broadcasted_iota(jnp.int32, (Q, K_TILE), 1) + k * K_TILE

    rv = run_vals[...]
    ri = run_idx[...]
    for _ in range(TOPK):
        m = jnp.max(s, axis=1, keepdims=True)
        cand = jnp.where(s == m, col, BIG_I32)
        a = jnp.min(cand, axis=1, keepdims=True)
        s = jnp.where(cand == a, NEG_INF, s)
        rv, ri = _merge_candidate(rv, ri, m, a)
    run_vals[...] = rv
    run_idx[...] = ri

    @pl.when(k == N_TILES - 1)
    def _finish():
        vals_out_ref[...] = rv
        idx_out_ref[...] = ri


@jax.jit
def kernel(Z, Y):
    vals, idx = pl.pallas_call(
        _topk_kernel,
        grid=(N_TILES,),
        in_specs=[
            pl.BlockSpec((Q, D), lambda k: (0, 0)),
            pl.BlockSpec((K_TILE, D), lambda k: (k, 0)),
        ],
        out_specs=[
            pl.BlockSpec((Q, TOPK), lambda k: (0, 0)),
            pl.BlockSpec((Q, TOPK), lambda k: (0, 0)),
        ],
        out_shape=[
            jax.ShapeDtypeStruct((Q, TOPK), jnp.float32),
            jax.ShapeDtypeStruct((Q, TOPK), jnp.int32),
        ],
        scratch_shapes=[
            pltpu.VMEM((Q, TOPK), jnp.float32),
            pltpu.VMEM((Q, TOPK), jnp.int32),
        ],
    )(Z, Y)
    return vals, idx


# speedup vs baseline: 2.4333x; 1.3565x over previous
import jax
import jax.numpy as jnp
from jax.experimental import pallas as pl
from jax.experimental.pallas import tpu as pltpu

Q = 1024
D = 512
K_TOTAL = 100000
K_TILE = 2000
N_TILES = K_TOTAL // K_TILE
TOPK = 5
LANES = 128
SUBL = 8
UNROLL = 5

NEG_INF = float("-inf")
BIG_I32 = 2**31 - 1


def _topk_kernel(z_ref, y_ref, vals_out_ref, idx_out_ref,
                 st_ref, run_v_ref, run_i_ref):
    k = pl.program_id(0)

    @pl.when(k == 0)
    def _init():
        run_v_ref[...] = jnp.full((SUBL, Q), NEG_INF, dtype=jnp.float32)
        run_i_ref[...] = jnp.zeros((SUBL, Q), dtype=jnp.int32)

    z = z_ref[...]
    zn = z / jnp.sqrt(jnp.sum(z * z, axis=1, keepdims=True))
    st_ref[...] = jax.lax.dot_general(
        y_ref[...], zn,
        dimension_numbers=(((1,), (1,)), ((), ())),
        preferred_element_type=jnp.float32,
    )

    base = k * K_TILE
    iota_s = jax.lax.broadcasted_iota(jnp.int32, (SUBL, LANES), 0)

    for g in range(Q // LANES):
        lo = g * LANES

        def body(r, carry, lo=lo):
            avs = list(carry[:TOPK])
            ais = list(carry[TOPK:])
            for u in range(UNROLL):
                row = (r * UNROLL + u) * SUBL
                v = st_ref[pl.ds(row, SUBL), lo:lo + LANES]
                iv = iota_s + (base + row)
                for j in range(TOPK):
                    gt = v > avs[j]
                    nav = jnp.where(gt, v, avs[j])
                    nai = jnp.where(gt, iv, ais[j])
                    v = jnp.where(gt, avs[j], v)
                    iv = jnp.where(gt, ais[j], iv)
                    avs[j] = nav
                    ais[j] = nai
            return tuple(avs) + tuple(ais)

        init = (tuple(jnp.full((SUBL, LANES), NEG_INF, dtype=jnp.float32)
                      for _ in range(TOPK))
                + tuple(jnp.zeros((SUBL, LANES), dtype=jnp.int32)
                        for _ in range(TOPK)))
        carry = jax.lax.fori_loop(0, K_TILE // SUBL // UNROLL, body, init,
                                  unroll=False)
        avs = list(carry[:TOPK])
        ais = list(carry[TOPK:])

        v_all = jnp.concatenate(avs + [run_v_ref[:, lo:lo + LANES]], axis=0)
        i_all = jnp.concatenate(ais + [run_i_ref[:, lo:lo + LANES]], axis=0)

        ms = []
        idxs = []
        for _ in range(TOPK):
            m = jnp.max(v_all, axis=0, keepdims=True)
            cand = jnp.where(v_all == m, i_all, BIG_I32)
            a = jnp.min(cand, axis=0, keepdims=True)
            v_all = jnp.where(cand == a, NEG_INF, v_all)
            ms.append(m)
            idxs.append(a)
        run_v_ref[:, lo:lo + LANES] = jnp.concatenate(
            ms + [ms[-1]] * (SUBL - TOPK), axis=0)
        run_i_ref[:, lo:lo + LANES] = jnp.concatenate(
            idxs + [idxs[-1]] * (SUBL - TOPK), axis=0)

    @pl.when(k == N_TILES - 1)
    def _finish():
        vals_out_ref[...] = run_v_ref[...]
        idx_out_ref[...] = run_i_ref[...]


@jax.jit
def kernel(Z, Y):
    vals_t, idx_t = pl.pallas_call(
        _topk_kernel,
        grid=(N_TILES,),
        in_specs=[
            pl.BlockSpec((Q, D), lambda k: (0, 0)),
            pl.BlockSpec((K_TILE, D), lambda k: (k, 0)),
        ],
        out_specs=[
            pl.BlockSpec((SUBL, Q), lambda k: (0, 0)),
            pl.BlockSpec((SUBL, Q), lambda k: (0, 0)),
        ],
        out_shape=[
            jax.ShapeDtypeStruct((SUBL, Q), jnp.float32),
            jax.ShapeDtypeStruct((SUBL, Q), jnp.int32),
        ],
        scratch_shapes=[
            pltpu.VMEM((K_TILE, Q), jnp.float32),
            pltpu.VMEM((SUBL, Q), jnp.float32),
            pltpu.VMEM((SUBL, Q), jnp.int32),
        ],
    )(Z, Y)
    return vals_t[:TOPK].T, idx_t[:TOPK].T


# speedup vs baseline: 2.8525x; 1.1723x over previous
import jax
import jax.numpy as jnp
from jax.experimental import pallas as pl
from jax.experimental.pallas import tpu as pltpu

Q = 1024
D = 512
K_TOTAL = 100000
K_TILE = 4000
N_TILES = K_TOTAL // K_TILE
TOPK = 5
LANES = 128
SUBL = 8
UNROLL = 10

NEG_INF = float("-inf")
BIG_I32 = 2**31 - 1


def _topk_kernel(z_ref, y_ref, vals_out_ref, idx_out_ref,
                 st_ref, run_v_ref, run_i_ref, zn_ref):
    k = pl.program_id(0)

    @pl.when(k == 0)
    def _init():
        run_v_ref[...] = jnp.full((SUBL, Q), NEG_INF, dtype=jnp.float32)
        run_i_ref[...] = jnp.zeros((SUBL, Q), dtype=jnp.int32)
        z = z_ref[...]
        zn_ref[...] = z / jnp.sqrt(jnp.sum(z * z, axis=1, keepdims=True))

    st_ref[...] = jax.lax.dot_general(
        y_ref[...], zn_ref[...],
        dimension_numbers=(((1,), (1,)), ((), ())),
        preferred_element_type=jnp.float32,
    )

    base = k * K_TILE
    iota_s = jax.lax.broadcasted_iota(jnp.int32, (SUBL, LANES), 0)

    for g in range(Q // LANES):
        lo = g * LANES

        def body(r, carry, lo=lo):
            avs = list(carry[:TOPK])
            ais = list(carry[TOPK:])
            for u in range(UNROLL):
                row = (r * UNROLL + u) * SUBL
                v = st_ref[pl.ds(row, SUBL), lo:lo + LANES]
                iv = iota_s + (base + row)
                for j in range(TOPK):
                    gt = v > avs[j]
                    nav = jnp.where(gt, v, avs[j])
                    nai = jnp.where(gt, iv, ais[j])
                    v = jnp.where(gt, avs[j], v)
                    iv = jnp.where(gt, ais[j], iv)
                    avs[j] = nav
                    ais[j] = nai
            return tuple(avs) + tuple(ais)

        init = (tuple(jnp.full((SUBL, LANES), NEG_INF, dtype=jnp.float32)
                      for _ in range(TOPK))
                + tuple(jnp.zeros((SUBL, LANES), dtype=jnp.int32)
                        for _ in range(TOPK)))
        carry = jax.lax.fori_loop(0, K_TILE // SUBL // UNROLL, body, init,
                                  unroll=False)
        avs = list(carry[:TOPK])
        ais = list(carry[TOPK:])

        v_all = jnp.concatenate(avs + [run_v_ref[:, lo:lo + LANES]], axis=0)
        i_all = jnp.concatenate(ais + [run_i_ref[:, lo:lo + LANES]], axis=0)

        ms = []
        idxs = []
        for _ in range(TOPK):
            m = jnp.max(v_all, axis=0, keepdims=True)
            cand = jnp.where(v_all == m, i_all, BIG_I32)
            a = jnp.min(cand, axis=0, keepdims=True)
            v_all = jnp.where(cand == a, NEG_INF, v_all)
            ms.append(m)
            idxs.append(a)
        run_v_ref[:, lo:lo + LANES] = jnp.concatenate(
            ms + [ms[-1]] * (SUBL - TOPK), axis=0)
        run_i_ref[:, lo:lo + LANES] = jnp.concatenate(
            idxs + [idxs[-1]] * (SUBL - TOPK), axis=0)

    @pl.when(k == N_TILES - 1)
    def _finish():
        vals_out_ref[...] = run_v_ref[...]
        idx_out_ref[...] = run_i_ref[...]


@jax.jit
def kernel(Z, Y):
    vals_t, idx_t = pl.pallas_call(
        _topk_kernel,
        grid=(N_TILES,),
        in_specs=[
            pl.BlockSpec((Q, D), lambda k: (0, 0)),
            pl.BlockSpec((K_TILE, D), lambda k: (k, 0)),
        ],
        out_specs=[
            pl.BlockSpec((SUBL, Q), lambda k: (0, 0)),
            pl.BlockSpec((SUBL, Q), lambda k: (0, 0)),
        ],
        out_shape=[
            jax.ShapeDtypeStruct((SUBL, Q), jnp.float32),
            jax.ShapeDtypeStruct((SUBL, Q), jnp.int32),
        ],
        scratch_shapes=[
            pltpu.VMEM((K_TILE, Q), jnp.float32),
            pltpu.VMEM((SUBL, Q), jnp.float32),
            pltpu.VMEM((SUBL, Q), jnp.int32),
            pltpu.VMEM((Q, D), jnp.float32),
        ],
    )(Z, Y)
    return vals_t[:TOPK].T, idx_t[:TOPK].T
